# baseline (speedup 1.0000x reference)
import math
from functools import partial

import jax
import jax.numpy as jnp
import numpy as np
from jax.experimental import pallas as pl

B, S, DIM = 1, 2048, 1024
NH, G, DK, DV = 16, 4, 64, 64
H = NH // G
L_CMP, D_STR, L_SEL, N_SEL, W_WIN = 32, 16, 64, 16, 512
NC = (S - L_CMP) // D_STR + 1
NCP = 128
NB = S // L_SEL
SCALE = 1.0 / DK ** 0.5
GH = DK // 2

TS = 256
TQ = 128
CK = 256
W_KV = W_WIN + TQ
NEG = -1e9

N_ROPE = NH * DK + 3 * G * DK
N_PROJ = NH * DK + 6 * G * DK


def _overlap_map_np():
    cs = np.arange(NC) * D_STR
    ce = cs + L_CMP
    ss = np.arange(NB) * L_SEL
    se = ss + L_SEL
    ov = np.clip(np.minimum(ce[:, None], se[None, :])
                 - np.maximum(cs[:, None], ss[None, :]), 0, None)
    m = (ov / float(L_CMP)).astype(np.float32)
    return np.concatenate([m, np.zeros((1, NB), np.float32)], axis=0)


def _proj_kernel(x_ref, w_ref, cosf_ref, sinf_ref,
                 q_ref, ks_ref, vs_ref, kw_ref, vw_ref, kc_ref, vc_ref):
    x = x_ref[...]
    h = jnp.dot(x, w_ref[...], preferred_element_type=jnp.float32)
    hr = h[:, :N_ROPE] * cosf_ref[...] + h[:, N_PROJ:] * sinf_ref[...]
    q = hr[:, :NH * DK]
    ks = hr[:, NH * DK: NH * DK + G * DK]
    kw = hr[:, NH * DK + G * DK: NH * DK + 2 * G * DK]
    kc = hr[:, NH * DK + 2 * G * DK: N_ROPE]
    vs = h[:, N_ROPE: N_ROPE + G * DV]
    vw = h[:, N_ROPE + G * DV: N_ROPE + 2 * G * DV]
    vc = h[:, N_ROPE + 2 * G * DV: N_PROJ]

    q_ref[...] = q
    ks_ref[...] = ks.astype(jnp.bfloat16)
    vs_ref[...] = vs.astype(jnp.bfloat16)
    kw_ref[...] = kw.astype(jnp.bfloat16)
    vw_ref[...] = vw.astype(jnp.bfloat16)

    nch = TS // D_STR
    r0 = jax.lax.broadcasted_iota(jnp.int32, (nch, TS), 0)
    r1 = jax.lax.broadcasted_iota(jnp.int32, (nch, TS), 1)
    smat = (r0 == r1 // D_STR).astype(jnp.float32)
    kc_ref[...] = jnp.dot(smat, kc, preferred_element_type=jnp.float32)
    vc_ref[...] = jnp.dot(smat, vc, preferred_element_type=jnp.float32)


def _softmax_last(s):
    m = jnp.max(s, axis=-1, keepdims=True)
    e = jnp.exp(s - m)
    return e / jnp.sum(e, axis=-1, keepdims=True)


def _attn_kernel(q_ref, ks_ref, vs_ref, kw_ref, vw_ref, kcs_ref, vcs_ref,
                 m_ref, gw1_ref, gb1_ref, gw2_ref, gb2_ref, wout_ref, out_ref):
    i = pl.program_id(0)
    g = pl.program_id(1)

    qs = q_ref[...]
    qh = qs.reshape(TQ, H, DK).transpose(1, 0, 2).reshape(H * TQ, DK)
    qh = qh * SCALE
    qh_b = qh.astype(jnp.bfloat16)

    kcs = kcs_ref[g]
    kc_next = jnp.concatenate([kcs[1:], kcs[:1]], axis=0)
    kcmp = (kcs + kc_next) * (1.0 / L_CMP)
    vcs = vcs_ref[g]
    vc_next = jnp.concatenate([vcs[1:], vcs[:1]], axis=0)
    vcmp = (vcs + vc_next) * (1.0 / L_CMP)

    sc = jnp.dot(qh, kcmp.T, preferred_element_type=jnp.float32)
    t_c = jax.lax.broadcasted_iota(jnp.int32, (TQ, NCP), 0) + i * TQ
    c_c = jax.lax.broadcasted_iota(jnp.int32, (TQ, NCP), 1)
    cmask = t_c >= (L_CMP - 1) + D_STR * c_c
    sc3 = sc.reshape(H, TQ, NCP)
    sc3 = jnp.where(cmask[None], sc3, NEG)
    p_cmp = _softmax_last(sc3)
    rowvalid = (t_c[:, :1] >= L_CMP - 1)
    p_cmp = jnp.where(rowvalid[None], p_cmp, 0.0)
    o_cmp = jnp.dot(p_cmp.reshape(H * TQ, NCP), vcmp,
                    preferred_element_type=jnp.float32)

    p_grp = jnp.dot(p_cmp.sum(axis=0), m_ref[...],
                    preferred_element_type=jnp.float32)
    t_b = jax.lax.broadcasted_iota(jnp.int32, (TQ, NB), 0) + i * TQ
    b_b = jax.lax.broadcasted_iota(jnp.int32, (TQ, NB), 1)
    forced = (b_b == 0) | (b_b == t_b // L_SEL)
    p_boost = p_grp + jnp.where(forced, 1e6, 0.0)
    pb_i = p_boost[:, :, None]
    pb_j = p_boost[:, None, :]
    j_ix = jax.lax.broadcasted_iota(jnp.int32, (TQ, NB, NB), 2)
    b_ix = jax.lax.broadcasted_iota(jnp.int32, (TQ, NB, NB), 1)
    beats = (pb_j > pb_i) | ((pb_j == pb_i) & (j_ix < b_ix))
    rank = jnp.sum(beats.astype(jnp.float32), axis=2)
    sel = (rank < N_SEL).astype(jnp.float32)

    def sel_body(c, carry):
        o_acc, d_acc = carry
        base = c * CK
        ks_c = ks_ref[g, pl.ds(base, CK), :]
        vs_c = vs_ref[g, pl.ds(base, CK), :]
        s = jnp.dot(qh_b, ks_c.T, preferred_element_type=jnp.float32)
        blk_b = jax.lax.broadcasted_iota(jnp.int32, (NB, CK), 0)
        pos_b = jax.lax.broadcasted_iota(jnp.int32, (NB, CK), 1) + base
        expand_c = (blk_b == pos_b // L_SEL).astype(jnp.float32)
        selpos = jnp.dot(sel, expand_c, preferred_element_type=jnp.float32)
        pos_q = jax.lax.broadcasted_iota(jnp.int32, (TQ, CK), 1) + base
        t_q = jax.lax.broadcasted_iota(jnp.int32, (TQ, CK), 0) + i * TQ
        msk = (selpos > 0.5) & (pos_q <= t_q)
        e = jnp.exp(jnp.where(msk[None], s.reshape(H, TQ, CK), NEG))
        e2 = e.reshape(H * TQ, CK)
        o_acc = o_acc + jnp.dot(e2.astype(jnp.bfloat16), vs_c,
                                preferred_element_type=jnp.float32)
        d_acc = d_acc + jnp.sum(e2, axis=1, keepdims=True)
        return o_acc, d_acc

    o0 = jnp.zeros((H * TQ, DV), jnp.float32)
    d0 = jnp.zeros((H * TQ, 1), jnp.float32)
    nchunks = i // (CK // TQ) + 1
    o_accs, d_accs = jax.lax.fori_loop(0, nchunks, sel_body, (o0, d0))
    o_sel = o_accs * (1.0 / d_accs)

    start = jnp.maximum(i - W_WIN // TQ, 0) * TQ
    kwin = kw_ref[g, pl.ds(start, W_KV), :]
    vwin = vw_ref[g, pl.ds(start, W_KV), :]
    sw = jnp.dot(qh_b, kwin.T, preferred_element_type=jnp.float32)
    t_w = jax.lax.broadcasted_iota(jnp.int32, (TQ, W_KV), 0) + i * TQ
    p_w = jax.lax.broadcasted_iota(jnp.int32, (TQ, W_KV), 1) + start
    wmask = (p_w <= t_w) & (p_w > t_w - W_WIN)
    ew = jnp.exp(jnp.where(wmask[None], sw.reshape(H, TQ, W_KV), NEG))
    ew2 = ew.reshape(H * TQ, W_KV)
    o_win = jnp.dot(ew2.astype(jnp.bfloat16), vwin,
                    preferred_element_type=jnp.float32)
    o_win = o_win * (1.0 / jnp.sum(ew2, axis=1, keepdims=True))

    q_gp = qs.reshape(TQ, H, DK).mean(axis=1)
    h1 = jnp.dot(q_gp, gw1_ref[...], preferred_element_type=jnp.float32) \
        + gb1_ref[...]
    h1 = h1 * jax.nn.sigmoid(h1)
    glog = jnp.dot(h1, gw2_ref[...], preferred_element_type=jnp.float32) \
        + gb2_ref[...]
    pg = _softmax_last(glog)
    a = glog[:, 0:1]
    b = glog[:, 1:2]
    c = glog[:, 2:3]
    m1 = jnp.maximum(a, jnp.maximum(b, c))
    ia0 = (a >= b) & (a >= c)
    ia1 = jnp.logical_not(ia0) & (b >= c)
    ia2 = jnp.logical_not(ia0) & jnp.logical_not(ia1)
    m2 = jnp.where(ia0, jnp.maximum(b, c),
                   jnp.where(ia1, jnp.maximum(a, c), jnp.maximum(a, b)))
    peaked = (m1 - m2) > 50.0
    p0 = jnp.where(peaked, ia0.astype(jnp.float32), pg[:, 0:1])
    p1 = jnp.where(peaked, ia1.astype(jnp.float32), pg[:, 1:2])
    p2 = jnp.where(peaked, ia2.astype(jnp.float32), pg[:, 2:3])

    o3 = (p0[None] * o_cmp.reshape(H, TQ, DV)
          + p1[None] * o_sel.reshape(H, TQ, DV)
          + p2[None] * o_win.reshape(H, TQ, DV))
    o = o3.transpose(1, 0, 2).reshape(TQ, H * DV)

    contrib = jnp.dot(o, wout_ref[g], preferred_element_type=jnp.float32)

    @pl.when(g == 0)
    def _():
        out_ref[...] = contrib

    @pl.when(g > 0)
    def _():
        out_ref[...] += contrib


def _swap_cols(w, n_heads):
    w4 = w.reshape(DIM, n_heads, 2, DK // 2)
    return jnp.concatenate([-w4[:, :, 1], w4[:, :, 0]], axis=2) \
        .reshape(DIM, n_heads * DK)


@jax.jit
def kernel(x, W_Q, W_K_sel, W_V_sel, W_K_win, W_V_win, W_K_cmp, W_V_cmp,
           W_out, g_w1, g_b1, g_w2, g_b2):
    xs = x.reshape(S, DIM)
    w_main = jnp.concatenate(
        [W_Q, W_K_sel, W_K_win, W_K_cmp, W_V_sel, W_V_win, W_V_cmp], axis=1)
    w_sw = jnp.concatenate(
        [_swap_cols(W_Q, NH), _swap_cols(W_K_sel, G),
         _swap_cols(W_K_win, G), _swap_cols(W_K_cmp, G)], axis=1)
    w_cat = jnp.concatenate([w_main, w_sw], axis=1)

    half = DK // 2
    inv = (10000.0 ** (-jnp.arange(half, dtype=jnp.float32) / half))
    ang = jnp.arange(S, dtype=jnp.float32)[:, None] * inv[None, :]
    cos2 = jnp.concatenate([jnp.cos(ang), jnp.cos(ang)], axis=1)
    sin2 = jnp.concatenate([jnp.sin(ang), jnp.sin(ang)], axis=1)
    cosf = jnp.tile(cos2, (1, N_ROPE // DK))
    sinf = jnp.tile(sin2, (1, N_ROPE // DK))

    nsb = S // TS
    nch = TS // D_STR
    proj_outs = pl.pallas_call(
        _proj_kernel,
        grid=(nsb,),
        in_specs=[
            pl.BlockSpec((TS, DIM), lambda i: (i, 0)),
            pl.BlockSpec((DIM, N_PROJ + N_ROPE), lambda i: (0, 0)),
            pl.BlockSpec((TS, N_ROPE), lambda i: (i, 0)),
            pl.BlockSpec((TS, N_ROPE), lambda i: (i, 0)),
        ],
        out_specs=[
            pl.BlockSpec((TS, NH * DK), lambda i: (i, 0)),
            pl.BlockSpec((TS, G * DK), lambda i: (i, 0)),
            pl.BlockSpec((TS, G * DV), lambda i: (i, 0)),
            pl.BlockSpec((TS, G * DK), lambda i: (i, 0)),
            pl.BlockSpec((TS, G * DV), lambda i: (i, 0)),
            pl.BlockSpec((nch, G * DK), lambda i: (i, 0)),
            pl.BlockSpec((nch, G * DV), lambda i: (i, 0)),
        ],
        out_shape=[
            jax.ShapeDtypeStruct((S, NH * DK), jnp.float32),
            jax.ShapeDtypeStruct((S, G * DK), jnp.bfloat16),
            jax.ShapeDtypeStruct((S, G * DV), jnp.bfloat16),
            jax.ShapeDtypeStruct((S, G * DK), jnp.bfloat16),
            jax.ShapeDtypeStruct((S, G * DV), jnp.bfloat16),
            jax.ShapeDtypeStruct((NCP, G * DK), jnp.float32),
            jax.ShapeDtypeStruct((NCP, G * DV), jnp.float32),
        ],
    )(xs, w_cat, cosf, sinf)
    q, ksel, vsel, kwin, vwin, kcsum, vcsum = proj_outs

    m_pad = jnp.asarray(_overlap_map_np())
    gw2_pad = jnp.concatenate(
        [g_w2, jnp.zeros((GH, 128 - 3), jnp.float32)], axis=1)
    gb2_pad = jnp.concatenate(
        [g_b2, jnp.full((128 - 3,), NEG, jnp.float32)]).reshape(1, 128)
    gb1_r = g_b1.reshape(1, GH)

    nqb = S // TQ
    out = pl.pallas_call(
        _attn_kernel,
        grid=(nqb, G),
        in_specs=[
            pl.BlockSpec((TQ, H * DK), lambda i, g: (i, g)),
            pl.BlockSpec((G, S, DK), lambda i, g: (0, 0, 0)),
            pl.BlockSpec((G, S, DV), lambda i, g: (0, 0, 0)),
            pl.BlockSpec((G, S, DK), lambda i, g: (0, 0, 0)),
            pl.BlockSpec((G, S, DV), lambda i, g: (0, 0, 0)),
            pl.BlockSpec((G, NCP, DK), lambda i, g: (0, 0, 0)),
            pl.BlockSpec((G, NCP, DV), lambda i, g: (0, 0, 0)),
            pl.BlockSpec((NCP, NB), lambda i, g: (0, 0)),
            pl.BlockSpec((DK, GH), lambda i, g: (0, 0)),
            pl.BlockSpec((1, GH), lambda i, g: (0, 0)),
            pl.BlockSpec((GH, 128), lambda i, g: (0, 0)),
            pl.BlockSpec((1, 128), lambda i, g: (0, 0)),
            pl.BlockSpec((G, H * DV, DIM), lambda i, g: (0, 0, 0)),
        ],
        out_specs=pl.BlockSpec((TQ, DIM), lambda i, g: (i, 0)),
        out_shape=jax.ShapeDtypeStruct((S, DIM), jnp.float32),
    )(
        q,
        ksel.reshape(S, G, DK).transpose(1, 0, 2),
        vsel.reshape(S, G, DV).transpose(1, 0, 2),
        kwin.reshape(S, G, DK).transpose(1, 0, 2),
        vwin.reshape(S, G, DV).transpose(1, 0, 2),
        kcsum.reshape(NCP, G, DK).transpose(1, 0, 2),
        vcsum.reshape(NCP, G, DV).transpose(1, 0, 2),
        m_pad, g_w1, gb1_r, gw2_pad, gb2_pad,
        W_out.reshape(G, H * DV, DIM),
    )
    return out.reshape(B, S, DIM)


# speedup vs baseline: 1.0332x; 1.0332x over previous
import math
from functools import partial

import jax
import jax.numpy as jnp
import numpy as np
from jax.experimental import pallas as pl

B, S, DIM = 1, 2048, 1024
NH, G, DK, DV = 16, 4, 64, 64
H = NH // G
L_CMP, D_STR, L_SEL, N_SEL, W_WIN = 32, 16, 64, 16, 512
NC = (S - L_CMP) // D_STR + 1
NCP = 128
NB = S // L_SEL
SCALE = 1.0 / DK ** 0.5
GH = DK // 2

TS = 256
TQ = 128
W_KV = W_WIN + TQ
NEG = -1e9


def _overlap_map_np():
    cs = np.arange(NC) * D_STR
    ce = cs + L_CMP
    ss = np.arange(NB) * L_SEL
    se = ss + L_SEL
    ov = np.clip(np.minimum(ce[:, None], se[None, :])
                 - np.maximum(cs[:, None], ss[None, :]), 0, None)
    m = (ov / float(L_CMP)).astype(np.float32)
    return np.concatenate([m, np.zeros((1, NB), np.float32)], axis=0)


def _rope_block(v, n_heads, cos, sin):
    v3 = v.reshape(TS, n_heads, DK)
    x1 = v3[..., : DK // 2]
    x2 = v3[..., DK // 2:]
    c = cos[:, None, :]
    s = sin[:, None, :]
    out = jnp.concatenate([x1 * c - x2 * s, x1 * s + x2 * c], axis=-1)
    return out.reshape(TS, n_heads * DK)


def _proj_kernel(x_ref, w_ref, q_ref, ks_ref, vs_ref, kw_ref, vw_ref,
                 kc_ref, vc_ref):
    i = pl.program_id(0)
    x = x_ref[...]
    h = jnp.dot(x, w_ref[...], preferred_element_type=jnp.float32)
    q = h[:, :NH * DK]
    ks = h[:, NH * DK + 0 * G * DK: NH * DK + 1 * G * DK]
    vs = h[:, NH * DK + 1 * G * DK: NH * DK + 2 * G * DK]
    kw = h[:, NH * DK + 2 * G * DK: NH * DK + 3 * G * DK]
    vw = h[:, NH * DK + 3 * G * DK: NH * DK + 4 * G * DK]
    kc = h[:, NH * DK + 4 * G * DK: NH * DK + 5 * G * DK]
    vc = h[:, NH * DK + 5 * G * DK: NH * DK + 6 * G * DK]

    half = DK // 2
    pos = (jax.lax.broadcasted_iota(jnp.int32, (TS, half), 0)
           + i * TS).astype(jnp.float32)
    fr = jax.lax.broadcasted_iota(jnp.int32, (TS, half), 1).astype(jnp.float32)
    inv = jnp.exp(fr * (-math.log(10000.0) / half))
    ang = pos * inv
    cos = jnp.cos(ang)
    sin = jnp.sin(ang)

    q_ref[...] = _rope_block(q, NH, cos, sin)
    ks_ref[...] = _rope_block(ks, G, cos, sin).astype(jnp.bfloat16)
    vs_ref[...] = vs.astype(jnp.bfloat16)
    kw_ref[...] = _rope_block(kw, G, cos, sin).astype(jnp.bfloat16)
    vw_ref[...] = vw.astype(jnp.bfloat16)
    kcr = _rope_block(kc, G, cos, sin)
    nch = TS // D_STR
    kc_ref[...] = kcr.reshape(nch, D_STR, G * DK).sum(axis=1)
    vc_ref[...] = vc.reshape(nch, D_STR, G * DK).sum(axis=1)


def _softmax_last(s):
    m = jnp.max(s, axis=-1, keepdims=True)
    e = jnp.exp(s - m)
    return e / jnp.sum(e, axis=-1, keepdims=True)


def _attn_kernel(q_ref, ks_ref, vs_ref, kw_ref, vw_ref, kcs_ref, vcs_ref,
                 m_ref, gw1_ref, gb1_ref, gw2_ref, gb2_ref, wout_ref, out_ref):
    i = pl.program_id(0)
    g = pl.program_id(1)

    qs = q_ref[0]
    qh = qs.reshape(TQ, H, DK).transpose(1, 0, 2).reshape(H * TQ, DK)
    qh = qh * SCALE
    qh_b = qh.astype(jnp.bfloat16)

    kcs = kcs_ref[g]
    kc_next = jnp.concatenate([kcs[1:], kcs[:1]], axis=0)
    kcmp = (kcs + kc_next) * (1.0 / L_CMP)
    vcs = vcs_ref[g]
    vc_next = jnp.concatenate([vcs[1:], vcs[:1]], axis=0)
    vcmp = (vcs + vc_next) * (1.0 / L_CMP)

    sc = jnp.dot(qh, kcmp.T, preferred_element_type=jnp.float32)
    t_c = jax.lax.broadcasted_iota(jnp.int32, (TQ, NCP), 0) + i * TQ
    c_c = jax.lax.broadcasted_iota(jnp.int32, (TQ, NCP), 1)
    cmask = t_c >= (L_CMP - 1) + D_STR * c_c
    sc3 = sc.reshape(H, TQ, NCP)
    sc3 = jnp.where(cmask[None], sc3, NEG)
    p_cmp = _softmax_last(sc3)
    rowvalid = (t_c[:, :1] >= L_CMP - 1)
    p_cmp = jnp.where(rowvalid[None], p_cmp, 0.0)
    o_cmp = jnp.dot(p_cmp.reshape(H * TQ, NCP), vcmp,
                    preferred_element_type=jnp.float32)

    p_grp = jnp.dot(p_cmp.sum(axis=0), m_ref[...],
                    preferred_element_type=jnp.float32)
    t_b = jax.lax.broadcasted_iota(jnp.int32, (TQ, NB), 0) + i * TQ
    b_b = jax.lax.broadcasted_iota(jnp.int32, (TQ, NB), 1)
    forced = (b_b == 0) | (b_b == t_b // L_SEL)
    p_boost = p_grp + jnp.where(forced, 1e6, 0.0)
    pb_i = p_boost[:, :, None]
    pb_j = p_boost[:, None, :]
    j_ix = jax.lax.broadcasted_iota(jnp.int32, (TQ, NB, NB), 2)
    b_ix = jax.lax.broadcasted_iota(jnp.int32, (TQ, NB, NB), 1)
    beats = (pb_j > pb_i) | ((pb_j == pb_i) & (j_ix < b_ix))
    rank = jnp.sum(beats.astype(jnp.float32), axis=2)
    sel = (rank < N_SEL).astype(jnp.float32)

    CK = 256

    def sel_body(c, carry):
        o_acc, d_acc = carry
        base = c * CK
        ks_c = ks_ref[g, pl.ds(base, CK), :]
        vs_c = vs_ref[g, pl.ds(base, CK), :]
        s = jnp.dot(qh_b, ks_c.T, preferred_element_type=jnp.float32)
        blk_b = jax.lax.broadcasted_iota(jnp.int32, (NB, CK), 0)
        pos_b = jax.lax.broadcasted_iota(jnp.int32, (NB, CK), 1) + base
        expand_c = (blk_b == pos_b // L_SEL).astype(jnp.float32)
        selpos = jnp.dot(sel, expand_c, preferred_element_type=jnp.float32)
        pos_q = jax.lax.broadcasted_iota(jnp.int32, (TQ, CK), 1) + base
        t_q = jax.lax.broadcasted_iota(jnp.int32, (TQ, CK), 0) + i * TQ
        msk = (selpos > 0.5) & (pos_q <= t_q)
        e = jnp.exp(jnp.where(msk[None], s.reshape(H, TQ, CK), NEG))
        e2 = e.reshape(H * TQ, CK)
        o_acc = o_acc + jnp.dot(e2.astype(jnp.bfloat16), vs_c,
                                preferred_element_type=jnp.float32)
        d_acc = d_acc + jnp.sum(e2, axis=1, keepdims=True)
        return o_acc, d_acc

    o0 = jnp.zeros((H * TQ, DV), jnp.float32)
    d0 = jnp.zeros((H * TQ, 1), jnp.float32)
    nchunks = i // (CK // TQ) + 1
    o_accs, d_accs = jax.lax.fori_loop(0, nchunks, sel_body, (o0, d0))
    o_sel = o_accs * (1.0 / d_accs)

    start = jnp.maximum(i - W_WIN // TQ, 0) * TQ
    kwin = kw_ref[g, pl.ds(start, W_KV), :]
    vwin = vw_ref[g, pl.ds(start, W_KV), :]
    sw = jnp.dot(qh_b, kwin.T, preferred_element_type=jnp.float32)
    t_w = jax.lax.broadcasted_iota(jnp.int32, (TQ, W_KV), 0) + i * TQ
    p_w = jax.lax.broadcasted_iota(jnp.int32, (TQ, W_KV), 1) + start
    wmask = (p_w <= t_w) & (p_w > t_w - W_WIN)
    ew = jnp.exp(jnp.where(wmask[None], sw.reshape(H, TQ, W_KV), NEG))
    ew2 = ew.reshape(H * TQ, W_KV)
    o_win = jnp.dot(ew2.astype(jnp.bfloat16), vwin,
                    preferred_element_type=jnp.float32)
    o_win = o_win * (1.0 / jnp.sum(ew2, axis=1, keepdims=True))

    q_gp = qs.reshape(TQ, H, DK).mean(axis=1)
    h1 = jnp.dot(q_gp, gw1_ref[...], preferred_element_type=jnp.float32) \
        + gb1_ref[...]
    h1 = h1 * jax.nn.sigmoid(h1)
    glog = jnp.dot(h1, gw2_ref[...], preferred_element_type=jnp.float32) \
        + gb2_ref[...]
    pg = _softmax_last(glog)
    a = glog[:, 0:1]
    b = glog[:, 1:2]
    c = glog[:, 2:3]
    m1 = jnp.maximum(a, jnp.maximum(b, c))
    ia0 = (a >= b) & (a >= c)
    ia1 = jnp.logical_not(ia0) & (b >= c)
    ia2 = jnp.logical_not(ia0) & jnp.logical_not(ia1)
    m2 = jnp.where(ia0, jnp.maximum(b, c),
                   jnp.where(ia1, jnp.maximum(a, c), jnp.maximum(a, b)))
    peaked = (m1 - m2) > 50.0
    p0 = jnp.where(peaked, ia0.astype(jnp.float32), pg[:, 0:1])
    p1 = jnp.where(peaked, ia1.astype(jnp.float32), pg[:, 1:2])
    p2 = jnp.where(peaked, ia2.astype(jnp.float32), pg[:, 2:3])

    o3 = (p0[None] * o_cmp.reshape(H, TQ, DV)
          + p1[None] * o_sel.reshape(H, TQ, DV)
          + p2[None] * o_win.reshape(H, TQ, DV))
    o = o3.transpose(1, 0, 2).reshape(TQ, H * DV)

    contrib = jnp.dot(o, wout_ref[g], preferred_element_type=jnp.float32)

    @pl.when(g == 0)
    def _():
        out_ref[...] = contrib

    @pl.when(g > 0)
    def _():
        out_ref[...] += contrib


@jax.jit
def kernel(x, W_Q, W_K_sel, W_V_sel, W_K_win, W_V_win, W_K_cmp, W_V_cmp,
           W_out, g_w1, g_b1, g_w2, g_b2):
    xs = x.reshape(S, DIM)
    w_all = jnp.concatenate(
        [W_Q, W_K_sel, W_V_sel, W_K_win, W_V_win, W_K_cmp, W_V_cmp], axis=1)

    nsb = S // TS
    nch = TS // D_STR
    proj_outs = pl.pallas_call(
        _proj_kernel,
        grid=(nsb,),
        in_specs=[
            pl.BlockSpec((TS, DIM), lambda i: (i, 0)),
            pl.BlockSpec((DIM, NH * DK + 6 * G * DK), lambda i: (0, 0)),
        ],
        out_specs=[
            pl.BlockSpec((TS, NH * DK), lambda i: (i, 0)),
            pl.BlockSpec((TS, G * DK), lambda i: (i, 0)),
            pl.BlockSpec((TS, G * DV), lambda i: (i, 0)),
            pl.BlockSpec((TS, G * DK), lambda i: (i, 0)),
            pl.BlockSpec((TS, G * DV), lambda i: (i, 0)),
            pl.BlockSpec((nch, G * DK), lambda i: (i, 0)),
            pl.BlockSpec((nch, G * DV), lambda i: (i, 0)),
        ],
        out_shape=[
            jax.ShapeDtypeStruct((S, NH * DK), jnp.float32),
            jax.ShapeDtypeStruct((S, G * DK), jnp.bfloat16),
            jax.ShapeDtypeStruct((S, G * DV), jnp.bfloat16),
            jax.ShapeDtypeStruct((S, G * DK), jnp.bfloat16),
            jax.ShapeDtypeStruct((S, G * DV), jnp.bfloat16),
            jax.ShapeDtypeStruct((NCP, G * DK), jnp.float32),
            jax.ShapeDtypeStruct((NCP, G * DV), jnp.float32),
        ],
    )(xs, w_all)
    q, ksel, vsel, kwin, vwin, kcsum, vcsum = proj_outs

    m_pad = jnp.asarray(_overlap_map_np())
    gw2_pad = jnp.concatenate(
        [g_w2, jnp.zeros((GH, 128 - 3), jnp.float32)], axis=1)
    gb2_pad = jnp.concatenate(
        [g_b2, jnp.full((128 - 3,), NEG, jnp.float32)]).reshape(1, 128)
    gb1_r = g_b1.reshape(1, GH)

    nqb = S // TQ
    out = pl.pallas_call(
        _attn_kernel,
        grid=(nqb, G),
        in_specs=[
            pl.BlockSpec((1, TQ, H * DK), lambda i, g: (g, i, 0)),
            pl.BlockSpec((G, S, DK), lambda i, g: (0, 0, 0)),
            pl.BlockSpec((G, S, DV), lambda i, g: (0, 0, 0)),
            pl.BlockSpec((G, S, DK), lambda i, g: (0, 0, 0)),
            pl.BlockSpec((G, S, DV), lambda i, g: (0, 0, 0)),
            pl.BlockSpec((G, NCP, DK), lambda i, g: (0, 0, 0)),
            pl.BlockSpec((G, NCP, DV), lambda i, g: (0, 0, 0)),
            pl.BlockSpec((NCP, NB), lambda i, g: (0, 0)),
            pl.BlockSpec((DK, GH), lambda i, g: (0, 0)),
            pl.BlockSpec((1, GH), lambda i, g: (0, 0)),
            pl.BlockSpec((GH, 128), lambda i, g: (0, 0)),
            pl.BlockSpec((1, 128), lambda i, g: (0, 0)),
            pl.BlockSpec((G, H * DV, DIM), lambda i, g: (0, 0, 0)),
        ],
        out_specs=pl.BlockSpec((TQ, DIM), lambda i, g: (i, 0)),
        out_shape=jax.ShapeDtypeStruct((S, DIM), jnp.float32),
    )(
        q.reshape(S, G, H * DK).transpose(1, 0, 2),
        ksel.reshape(S, G, DK).transpose(1, 0, 2),
        vsel.reshape(S, G, DV).transpose(1, 0, 2),
        kwin.reshape(S, G, DK).transpose(1, 0, 2),
        vwin.reshape(S, G, DV).transpose(1, 0, 2),
        kcsum.reshape(NCP, G, DK).transpose(1, 0, 2),
        vcsum.reshape(NCP, G, DV).transpose(1, 0, 2),
        m_pad, g_w1, gb1_r, gw2_pad, gb2_pad,
        W_out.reshape(G, H * DV, DIM),
    )
    return out.reshape(B, S, DIM)


# speedup vs baseline: 1.1146x; 1.0787x over previous
import math
from functools import partial

import jax
import jax.numpy as jnp
import numpy as np
from jax.experimental import pallas as pl

B, S, DIM = 1, 2048, 1024
NH, G, DK, DV = 16, 4, 64, 64
H = NH // G
L_CMP, D_STR, L_SEL, N_SEL, W_WIN = 32, 16, 64, 16, 512
NC = (S - L_CMP) // D_STR + 1
NCP = 128
NB = S // L_SEL
SCALE = 1.0 / DK ** 0.5
GH = DK // 2

TS = 256
TQ = 128
CK = 256
W_KV = W_WIN + TQ
NEG = -1e9


def _overlap_map_np():
    cs = np.arange(NC) * D_STR
    ce = cs + L_CMP
    ss = np.arange(NB) * L_SEL
    se = ss + L_SEL
    ov = np.clip(np.minimum(ce[:, None], se[None, :])
                 - np.maximum(cs[:, None], ss[None, :]), 0, None)
    m = (ov / float(L_CMP)).astype(np.float32)
    return np.concatenate([m, np.zeros((1, NB), np.float32)], axis=0)


def _rope_block(v, n_heads, cos, sin):
    v3 = v.reshape(TS, n_heads, DK)
    x1 = v3[..., : DK // 2]
    x2 = v3[..., DK // 2:]
    c = cos[:, None, :]
    s = sin[:, None, :]
    out = jnp.concatenate([x1 * c - x2 * s, x1 * s + x2 * c], axis=-1)
    return out.reshape(TS, n_heads * DK)


def _proj_kernel(x_ref, w_ref, q_ref, ks_ref, vs_ref, kw_ref, vw_ref,
                 kc_ref, vc_ref):
    i = pl.program_id(0)
    x = x_ref[...]
    h = jnp.dot(x, w_ref[...], preferred_element_type=jnp.float32)
    q = h[:, :NH * DK]
    ks = h[:, NH * DK + 0 * G * DK: NH * DK + 1 * G * DK]
    vs = h[:, NH * DK + 1 * G * DK: NH * DK + 2 * G * DK]
    kw = h[:, NH * DK + 2 * G * DK: NH * DK + 3 * G * DK]
    vw = h[:, NH * DK + 3 * G * DK: NH * DK + 4 * G * DK]
    kc = h[:, NH * DK + 4 * G * DK: NH * DK + 5 * G * DK]
    vc = h[:, NH * DK + 5 * G * DK: NH * DK + 6 * G * DK]

    half = DK // 2
    pos = (jax.lax.broadcasted_iota(jnp.int32, (TS, half), 0)
           + i * TS).astype(jnp.float32)
    fr = jax.lax.broadcasted_iota(jnp.int32, (TS, half), 1).astype(jnp.float32)
    inv = jnp.exp(fr * (-math.log(10000.0) / half))
    ang = pos * inv
    cos = jnp.cos(ang)
    sin = jnp.sin(ang)

    q_ref[...] = _rope_block(q, NH, cos, sin)
    ks_ref[...] = _rope_block(ks, G, cos, sin).astype(jnp.bfloat16)
    vs_ref[...] = vs.astype(jnp.bfloat16)
    kw_ref[...] = _rope_block(kw, G, cos, sin).astype(jnp.bfloat16)
    vw_ref[...] = vw.astype(jnp.bfloat16)
    kcr = _rope_block(kc, G, cos, sin)
    nch = TS // D_STR
    kc_ref[...] = kcr.reshape(nch, D_STR, G * DK).sum(axis=1)
    vc_ref[...] = vc.reshape(nch, D_STR, G * DK).sum(axis=1)


def _softmax_last(s):
    m = jnp.max(s, axis=-1, keepdims=True)
    e = jnp.exp(s - m)
    return e / jnp.sum(e, axis=-1, keepdims=True)


def _attn_kernel(q_ref, ks_ref, vs_ref, kw_ref, vw_ref, kcs_ref, vcs_ref,
                 m_ref, gw1_ref, gb1_ref, gw2_ref, gb2_ref, wout_ref, out_ref):
    i = pl.program_id(0)
    qs_all = q_ref[...]

    acc = None
    for g in range(G):
        qs = qs_all[:, g * H * DK:(g + 1) * H * DK]
        qh = qs.reshape(TQ, H, DK).transpose(1, 0, 2).reshape(H * TQ, DK)
        qh = qh * SCALE
        qh_b = qh.astype(jnp.bfloat16)

        kcs = kcs_ref[:, g * DK:(g + 1) * DK]
        kc_next = jnp.concatenate([kcs[1:], kcs[:1]], axis=0)
        kcmp = (kcs + kc_next) * (1.0 / L_CMP)
        vcs = vcs_ref[:, g * DV:(g + 1) * DV]
        vc_next = jnp.concatenate([vcs[1:], vcs[:1]], axis=0)
        vcmp = (vcs + vc_next) * (1.0 / L_CMP)

        sc = jnp.dot(qh, kcmp.T, preferred_element_type=jnp.float32)
        t_c = jax.lax.broadcasted_iota(jnp.int32, (TQ, NCP), 0) + i * TQ
        c_c = jax.lax.broadcasted_iota(jnp.int32, (TQ, NCP), 1)
        cmask = t_c >= (L_CMP - 1) + D_STR * c_c
        sc3 = jnp.where(cmask[None], sc.reshape(H, TQ, NCP), NEG)
        p_cmp = _softmax_last(sc3)
        rowvalid = (t_c[:, :1] >= L_CMP - 1)
        p_cmp = jnp.where(rowvalid[None], p_cmp, 0.0)
        o_cmp = jnp.dot(p_cmp.reshape(H * TQ, NCP), vcmp,
                        preferred_element_type=jnp.float32)

        p_grp = jnp.dot(p_cmp.sum(axis=0), m_ref[...],
                        preferred_element_type=jnp.float32)
        t_b = jax.lax.broadcasted_iota(jnp.int32, (TQ, NB), 0) + i * TQ
        b_b = jax.lax.broadcasted_iota(jnp.int32, (TQ, NB), 1)
        forced = (b_b == 0) | (b_b == t_b // L_SEL)
        p_boost = p_grp + jnp.where(forced, 1e6, 0.0)
        pb_i = p_boost[:, :, None]
        pb_j = p_boost[:, None, :]
        j_ix = jax.lax.broadcasted_iota(jnp.int32, (TQ, NB, NB), 2)
        b_ix = jax.lax.broadcasted_iota(jnp.int32, (TQ, NB, NB), 1)
        beats = (pb_j > pb_i) | ((pb_j == pb_i) & (j_ix < b_ix))
        rank = jnp.sum(beats.astype(jnp.float32), axis=2)
        sel = (rank < N_SEL).astype(jnp.float32)

        def sel_body(c, carry, g=g, qh_b=qh_b, sel=sel):
            o_acc, d_acc = carry
            base = c * CK
            ks_c = ks_ref[pl.ds(base, CK), g * DK:(g + 1) * DK]
            vs_c = vs_ref[pl.ds(base, CK), g * DV:(g + 1) * DV]
            s = jnp.dot(qh_b, ks_c.T, preferred_element_type=jnp.float32)
            blk_b = jax.lax.broadcasted_iota(jnp.int32, (NB, CK), 0)
            pos_b = jax.lax.broadcasted_iota(jnp.int32, (NB, CK), 1) + base
            expand_c = (blk_b == pos_b // L_SEL).astype(jnp.float32)
            selpos = jnp.dot(sel, expand_c, preferred_element_type=jnp.float32)
            pos_q = jax.lax.broadcasted_iota(jnp.int32, (TQ, CK), 1) + base
            t_q = jax.lax.broadcasted_iota(jnp.int32, (TQ, CK), 0) + i * TQ
            msk = (selpos > 0.5) & (pos_q <= t_q)
            e = jnp.exp(jnp.where(msk[None], s.reshape(H, TQ, CK), NEG))
            e2 = e.reshape(H * TQ, CK)
            o_acc = o_acc + jnp.dot(e2.astype(jnp.bfloat16), vs_c,
                                    preferred_element_type=jnp.float32)
            d_acc = d_acc + jnp.sum(e2, axis=1, keepdims=True)
            return o_acc, d_acc

        o0 = jnp.zeros((H * TQ, DV), jnp.float32)
        d0 = jnp.zeros((H * TQ, 1), jnp.float32)
        nchunks = i // (CK // TQ) + 1
        o_accs, d_accs = jax.lax.fori_loop(0, nchunks, sel_body, (o0, d0))
        o_sel = o_accs * (1.0 / d_accs)

        start = jnp.maximum(i - W_WIN // TQ, 0) * TQ
        kwin = kw_ref[pl.ds(start, W_KV), g * DK:(g + 1) * DK]
        vwin = vw_ref[pl.ds(start, W_KV), g * DV:(g + 1) * DV]
        sw = jnp.dot(qh_b, kwin.T, preferred_element_type=jnp.float32)
        t_w = jax.lax.broadcasted_iota(jnp.int32, (TQ, W_KV), 0) + i * TQ
        p_w = jax.lax.broadcasted_iota(jnp.int32, (TQ, W_KV), 1) + start
        wmask = (p_w <= t_w) & (p_w > t_w - W_WIN)
        ew = jnp.exp(jnp.where(wmask[None], sw.reshape(H, TQ, W_KV), NEG))
        ew2 = ew.reshape(H * TQ, W_KV)
        o_win = jnp.dot(ew2.astype(jnp.bfloat16), vwin,
                        preferred_element_type=jnp.float32)
        o_win = o_win * (1.0 / jnp.sum(ew2, axis=1, keepdims=True))

        q_gp = qs.reshape(TQ, H, DK).mean(axis=1)
        h1 = jnp.dot(q_gp, gw1_ref[...],
                     preferred_element_type=jnp.float32) + gb1_ref[...]
        h1 = h1 * jax.nn.sigmoid(h1)
        glog = jnp.dot(h1, gw2_ref[...],
                       preferred_element_type=jnp.float32) + gb2_ref[...]
        pg = _softmax_last(glog)
        a = glog[:, 0:1]
        b = glog[:, 1:2]
        c = glog[:, 2:3]
        m1 = jnp.maximum(a, jnp.maximum(b, c))
        ia0 = (a >= b) & (a >= c)
        ia1 = jnp.logical_not(ia0) & (b >= c)
        ia2 = jnp.logical_not(ia0) & jnp.logical_not(ia1)
        m2 = jnp.where(ia0, jnp.maximum(b, c),
                       jnp.where(ia1, jnp.maximum(a, c), jnp.maximum(a, b)))
        peaked = (m1 - m2) > 50.0
        p0 = jnp.where(peaked, ia0.astype(jnp.float32), pg[:, 0:1])
        p1 = jnp.where(peaked, ia1.astype(jnp.float32), pg[:, 1:2])
        p2 = jnp.where(peaked, ia2.astype(jnp.float32), pg[:, 2:3])

        o3 = (p0[None] * o_cmp.reshape(H, TQ, DV)
              + p1[None] * o_sel.reshape(H, TQ, DV)
              + p2[None] * o_win.reshape(H, TQ, DV))
        o = o3.transpose(1, 0, 2).reshape(TQ, H * DV)

        w_g = wout_ref[g * H * DV:(g + 1) * H * DV, :]
        contrib = jnp.dot(o, w_g, preferred_element_type=jnp.float32)
        acc = contrib if acc is None else acc + contrib

    out_ref[...] = acc


@jax.jit
def kernel(x, W_Q, W_K_sel, W_V_sel, W_K_win, W_V_win, W_K_cmp, W_V_cmp,
           W_out, g_w1, g_b1, g_w2, g_b2):
    xs = x.reshape(S, DIM)
    w_all = jnp.concatenate(
        [W_Q, W_K_sel, W_V_sel, W_K_win, W_V_win, W_K_cmp, W_V_cmp], axis=1)

    nsb = S // TS
    nch = TS // D_STR
    proj_outs = pl.pallas_call(
        _proj_kernel,
        grid=(nsb,),
        in_specs=[
            pl.BlockSpec((TS, DIM), lambda i: (i, 0)),
            pl.BlockSpec((DIM, NH * DK + 6 * G * DK), lambda i: (0, 0)),
        ],
        out_specs=[
            pl.BlockSpec((TS, NH * DK), lambda i: (i, 0)),
            pl.BlockSpec((TS, G * DK), lambda i: (i, 0)),
            pl.BlockSpec((TS, G * DV), lambda i: (i, 0)),
            pl.BlockSpec((TS, G * DK), lambda i: (i, 0)),
            pl.BlockSpec((TS, G * DV), lambda i: (i, 0)),
            pl.BlockSpec((nch, G * DK), lambda i: (i, 0)),
            pl.BlockSpec((nch, G * DV), lambda i: (i, 0)),
        ],
        out_shape=[
            jax.ShapeDtypeStruct((S, NH * DK), jnp.float32),
            jax.ShapeDtypeStruct((S, G * DK), jnp.bfloat16),
            jax.ShapeDtypeStruct((S, G * DV), jnp.bfloat16),
            jax.ShapeDtypeStruct((S, G * DK), jnp.bfloat16),
            jax.ShapeDtypeStruct((S, G * DV), jnp.bfloat16),
            jax.ShapeDtypeStruct((NCP, G * DK), jnp.float32),
            jax.ShapeDtypeStruct((NCP, G * DV), jnp.float32),
        ],
    )(xs, w_all)
    q, ksel, vsel, kwin, vwin, kcsum, vcsum = proj_outs

    m_pad = jnp.asarray(_overlap_map_np())
    gw2_pad = jnp.concatenate(
        [g_w2, jnp.zeros((GH, 128 - 3), jnp.float32)], axis=1)
    gb2_pad = jnp.concatenate(
        [g_b2, jnp.full((128 - 3,), NEG, jnp.float32)]).reshape(1, 128)
    gb1_r = g_b1.reshape(1, GH)

    nqb = S // TQ
    out = pl.pallas_call(
        _attn_kernel,
        grid=(nqb,),
        in_specs=[
            pl.BlockSpec((TQ, NH * DK), lambda i: (i, 0)),
            pl.BlockSpec((S, G * DK), lambda i: (0, 0)),
            pl.BlockSpec((S, G * DV), lambda i: (0, 0)),
            pl.BlockSpec((S, G * DK), lambda i: (0, 0)),
            pl.BlockSpec((S, G * DV), lambda i: (0, 0)),
            pl.BlockSpec((NCP, G * DK), lambda i: (0, 0)),
            pl.BlockSpec((NCP, G * DV), lambda i: (0, 0)),
            pl.BlockSpec((NCP, NB), lambda i: (0, 0)),
            pl.BlockSpec((DK, GH), lambda i: (0, 0)),
            pl.BlockSpec((1, GH), lambda i: (0, 0)),
            pl.BlockSpec((GH, 128), lambda i: (0, 0)),
            pl.BlockSpec((1, 128), lambda i: (0, 0)),
            pl.BlockSpec((NH * DV, DIM), lambda i: (0, 0)),
        ],
        out_specs=pl.BlockSpec((TQ, DIM), lambda i: (i, 0)),
        out_shape=jax.ShapeDtypeStruct((S, DIM), jnp.float32),
    )(q, ksel, vsel, kwin, vwin, kcsum, vcsum,
      m_pad, g_w1, gb1_r, gw2_pad, gb2_pad, W_out)
    return out.reshape(B, S, DIM)


# speedup vs baseline: 1.2477x; 1.1195x over previous
import math
from functools import partial

import jax
import jax.numpy as jnp
import numpy as np
from jax.experimental import pallas as pl

B, S, DIM = 1, 2048, 1024
NH, G, DK, DV = 16, 4, 64, 64
H = NH // G
L_CMP, D_STR, L_SEL, N_SEL, W_WIN = 32, 16, 64, 16, 512
NC = (S - L_CMP) // D_STR + 1
NCP = 128
NB = S // L_SEL
SCALE = 1.0 / DK ** 0.5
GH = DK // 2

TS = 256
TQ = 256
CK = 512
W_KV = W_WIN + TQ
NEG = -1e9


def _overlap_map_np():
    cs = np.arange(NC) * D_STR
    ce = cs + L_CMP
    ss = np.arange(NB) * L_SEL
    se = ss + L_SEL
    ov = np.clip(np.minimum(ce[:, None], se[None, :])
                 - np.maximum(cs[:, None], ss[None, :]), 0, None)
    m = (ov / float(L_CMP)).astype(np.float32)
    return np.concatenate([m, np.zeros((1, NB), np.float32)], axis=0)


def _rope_block(v, n_heads, cos, sin):
    v3 = v.reshape(TS, n_heads, DK)
    x1 = v3[..., : DK // 2]
    x2 = v3[..., DK // 2:]
    c = cos[:, None, :]
    s = sin[:, None, :]
    out = jnp.concatenate([x1 * c - x2 * s, x1 * s + x2 * c], axis=-1)
    return out.reshape(TS, n_heads * DK)


def _proj_kernel(x_ref, w_ref, q_ref, ks_ref, vs_ref, kw_ref, vw_ref,
                 kc_ref, vc_ref):
    i = pl.program_id(0)
    x = x_ref[...]
    h = jnp.dot(x, w_ref[...], preferred_element_type=jnp.float32)
    q = h[:, :NH * DK]
    ks = h[:, NH * DK + 0 * G * DK: NH * DK + 1 * G * DK]
    vs = h[:, NH * DK + 1 * G * DK: NH * DK + 2 * G * DK]
    kw = h[:, NH * DK + 2 * G * DK: NH * DK + 3 * G * DK]
    vw = h[:, NH * DK + 3 * G * DK: NH * DK + 4 * G * DK]
    kc = h[:, NH * DK + 4 * G * DK: NH * DK + 5 * G * DK]
    vc = h[:, NH * DK + 5 * G * DK: NH * DK + 6 * G * DK]

    half = DK // 2
    pos = (jax.lax.broadcasted_iota(jnp.int32, (TS, half), 0)
           + i * TS).astype(jnp.float32)
    fr = jax.lax.broadcasted_iota(jnp.int32, (TS, half), 1).astype(jnp.float32)
    inv = jnp.exp(fr * (-math.log(10000.0) / half))
    ang = pos * inv
    cos = jnp.cos(ang)
    sin = jnp.sin(ang)

    q_ref[...] = _rope_block(q, NH, cos, sin)
    ks_ref[...] = _rope_block(ks, G, cos, sin).astype(jnp.bfloat16)
    vs_ref[...] = vs.astype(jnp.bfloat16)
    kw_ref[...] = _rope_block(kw, G, cos, sin).astype(jnp.bfloat16)
    vw_ref[...] = vw.astype(jnp.bfloat16)
    kcr = _rope_block(kc, G, cos, sin)
    nch = TS // D_STR
    kc_ref[...] = kcr.reshape(nch, D_STR, G * DK).sum(axis=1)
    vc_ref[...] = vc.reshape(nch, D_STR, G * DK).sum(axis=1)


def _softmax_last(s):
    m = jnp.max(s, axis=-1, keepdims=True)
    e = jnp.exp(s - m)
    return e / jnp.sum(e, axis=-1, keepdims=True)


def _attn_kernel(q_ref, ks_ref, vs_ref, kw_ref, vw_ref, kcs_ref, vcs_ref,
                 m_ref, ex_ref, gw1_ref, gb1_ref, gw2_ref, gb2_ref, wout_ref,
                 out_ref):
    i = pl.program_id(0)
    qs_all = q_ref[...]

    acc = None
    for g in range(G):
        qs = qs_all[:, g * H * DK:(g + 1) * H * DK]
        qh = qs.reshape(TQ, H, DK).transpose(1, 0, 2).reshape(H * TQ, DK)
        qh = qh * SCALE
        qh_b = qh.astype(jnp.bfloat16)

        kcs = kcs_ref[:, g * DK:(g + 1) * DK]
        kc_next = jnp.concatenate([kcs[1:], kcs[:1]], axis=0)
        kcmp = (kcs + kc_next) * (1.0 / L_CMP)
        vcs = vcs_ref[:, g * DV:(g + 1) * DV]
        vc_next = jnp.concatenate([vcs[1:], vcs[:1]], axis=0)
        vcmp = (vcs + vc_next) * (1.0 / L_CMP)

        sc = jnp.dot(qh, kcmp.T, preferred_element_type=jnp.float32)
        t_c = jax.lax.broadcasted_iota(jnp.int32, (TQ, NCP), 0) + i * TQ
        c_c = jax.lax.broadcasted_iota(jnp.int32, (TQ, NCP), 1)
        cmask = t_c >= (L_CMP - 1) + D_STR * c_c
        sc3 = jnp.where(cmask[None], sc.reshape(H, TQ, NCP), NEG)
        p_cmp = _softmax_last(sc3)
        rowvalid = (t_c[:, :1] >= L_CMP - 1)
        p_cmp = jnp.where(rowvalid[None], p_cmp, 0.0)
        o_cmp = jnp.dot(p_cmp.reshape(H * TQ, NCP), vcmp,
                        preferred_element_type=jnp.float32)

        p_grp = jnp.dot(p_cmp.sum(axis=0), m_ref[...],
                        preferred_element_type=jnp.float32)
        t_b = jax.lax.broadcasted_iota(jnp.int32, (TQ, NB), 0) + i * TQ
        b_b = jax.lax.broadcasted_iota(jnp.int32, (TQ, NB), 1)
        forced = (b_b == 0) | (b_b == t_b // L_SEL)
        p_boost = p_grp + jnp.where(forced, 1e6, 0.0)
        pb_i = p_boost[:, :, None]
        pb_j = p_boost[:, None, :]
        j_ix = jax.lax.broadcasted_iota(jnp.int32, (TQ, NB, NB), 2)
        b_ix = jax.lax.broadcasted_iota(jnp.int32, (TQ, NB, NB), 1)
        beats = (pb_j > pb_i) | ((pb_j == pb_i) & (j_ix < b_ix))
        rank = jnp.sum(beats.astype(jnp.float32), axis=2)
        sel = (rank < N_SEL).astype(jnp.float32)

        def sel_body(c, carry, g=g, qh_b=qh_b, sel=sel):
            o_acc, d_acc = carry
            base = c * CK
            ks_c = ks_ref[pl.ds(base, CK), g * DK:(g + 1) * DK]
            vs_c = vs_ref[pl.ds(base, CK), g * DV:(g + 1) * DV]
            s = jnp.dot(qh_b, ks_c.T, preferred_element_type=jnp.float32)
            expand_c = ex_ref[:, pl.ds(base, CK)]
            selpos = jnp.dot(sel, expand_c, preferred_element_type=jnp.float32)
            pos_q = jax.lax.broadcasted_iota(jnp.int32, (TQ, CK), 1) + base
            t_q = jax.lax.broadcasted_iota(jnp.int32, (TQ, CK), 0) + i * TQ
            msk = (selpos > 0.5) & (pos_q <= t_q)
            e = jnp.exp(jnp.where(msk[None], s.reshape(H, TQ, CK), NEG))
            e2 = e.reshape(H * TQ, CK)
            o_acc = o_acc + jnp.dot(e2.astype(jnp.bfloat16), vs_c,
                                    preferred_element_type=jnp.float32)
            d_acc = d_acc + jnp.sum(e2, axis=1, keepdims=True)
            return o_acc, d_acc

        o0 = jnp.zeros((H * TQ, DV), jnp.float32)
        d0 = jnp.zeros((H * TQ, 1), jnp.float32)
        nchunks = i // (CK // TQ) + 1
        o_accs, d_accs = jax.lax.fori_loop(0, nchunks, sel_body, (o0, d0))
        o_sel = o_accs * (1.0 / d_accs)

        start = jnp.maximum(i - W_WIN // TQ, 0) * TQ
        kwin = kw_ref[pl.ds(start, W_KV), g * DK:(g + 1) * DK]
        vwin = vw_ref[pl.ds(start, W_KV), g * DV:(g + 1) * DV]
        sw = jnp.dot(qh_b, kwin.T, preferred_element_type=jnp.float32)
        t_w = jax.lax.broadcasted_iota(jnp.int32, (TQ, W_KV), 0) + i * TQ
        p_w = jax.lax.broadcasted_iota(jnp.int32, (TQ, W_KV), 1) + start
        wmask = (p_w <= t_w) & (p_w > t_w - W_WIN)
        ew = jnp.exp(jnp.where(wmask[None], sw.reshape(H, TQ, W_KV), NEG))
        ew2 = ew.reshape(H * TQ, W_KV)
        o_win = jnp.dot(ew2.astype(jnp.bfloat16), vwin,
                        preferred_element_type=jnp.float32)
        o_win = o_win * (1.0 / jnp.sum(ew2, axis=1, keepdims=True))

        q_gp = qs.reshape(TQ, H, DK).mean(axis=1)
        h1 = jnp.dot(q_gp, gw1_ref[...],
                     preferred_element_type=jnp.float32) + gb1_ref[...]
        h1 = h1 * jax.nn.sigmoid(h1)
        glog = jnp.dot(h1, gw2_ref[...],
                       preferred_element_type=jnp.float32) + gb2_ref[...]
        pg = _softmax_last(glog)
        a = glog[:, 0:1]
        b = glog[:, 1:2]
        c = glog[:, 2:3]
        m1 = jnp.maximum(a, jnp.maximum(b, c))
        ia0 = (a >= b) & (a >= c)
        ia1 = jnp.logical_not(ia0) & (b >= c)
        ia2 = jnp.logical_not(ia0) & jnp.logical_not(ia1)
        m2 = jnp.where(ia0, jnp.maximum(b, c),
                       jnp.where(ia1, jnp.maximum(a, c), jnp.maximum(a, b)))
        peaked = (m1 - m2) > 50.0
        p0 = jnp.where(peaked, ia0.astype(jnp.float32), pg[:, 0:1])
        p1 = jnp.where(peaked, ia1.astype(jnp.float32), pg[:, 1:2])
        p2 = jnp.where(peaked, ia2.astype(jnp.float32), pg[:, 2:3])

        o3 = (p0[None] * o_cmp.reshape(H, TQ, DV)
              + p1[None] * o_sel.reshape(H, TQ, DV)
              + p2[None] * o_win.reshape(H, TQ, DV))
        o = o3.transpose(1, 0, 2).reshape(TQ, H * DV)

        w_g = wout_ref[g * H * DV:(g + 1) * H * DV, :]
        contrib = jnp.dot(o, w_g, preferred_element_type=jnp.float32)
        acc = contrib if acc is None else acc + contrib

    out_ref[...] = acc


@jax.jit
def kernel(x, W_Q, W_K_sel, W_V_sel, W_K_win, W_V_win, W_K_cmp, W_V_cmp,
           W_out, g_w1, g_b1, g_w2, g_b2):
    xs = x.reshape(S, DIM)
    w_all = jnp.concatenate(
        [W_Q, W_K_sel, W_V_sel, W_K_win, W_V_win, W_K_cmp, W_V_cmp], axis=1)

    nsb = S // TS
    nch = TS // D_STR
    proj_outs = pl.pallas_call(
        _proj_kernel,
        grid=(nsb,),
        in_specs=[
            pl.BlockSpec((TS, DIM), lambda i: (i, 0)),
            pl.BlockSpec((DIM, NH * DK + 6 * G * DK), lambda i: (0, 0)),
        ],
        out_specs=[
            pl.BlockSpec((TS, NH * DK), lambda i: (i, 0)),
            pl.BlockSpec((TS, G * DK), lambda i: (i, 0)),
            pl.BlockSpec((TS, G * DV), lambda i: (i, 0)),
            pl.BlockSpec((TS, G * DK), lambda i: (i, 0)),
            pl.BlockSpec((TS, G * DV), lambda i: (i, 0)),
            pl.BlockSpec((nch, G * DK), lambda i: (i, 0)),
            pl.BlockSpec((nch, G * DV), lambda i: (i, 0)),
        ],
        out_shape=[
            jax.ShapeDtypeStruct((S, NH * DK), jnp.float32),
            jax.ShapeDtypeStruct((S, G * DK), jnp.bfloat16),
            jax.ShapeDtypeStruct((S, G * DV), jnp.bfloat16),
            jax.ShapeDtypeStruct((S, G * DK), jnp.bfloat16),
            jax.ShapeDtypeStruct((S, G * DV), jnp.bfloat16),
            jax.ShapeDtypeStruct((NCP, G * DK), jnp.float32),
            jax.ShapeDtypeStruct((NCP, G * DV), jnp.float32),
        ],
    )(xs, w_all)
    q, ksel, vsel, kwin, vwin, kcsum, vcsum = proj_outs

    m_pad = jnp.asarray(_overlap_map_np())
    expand_all = jnp.asarray(
        (np.arange(NB)[:, None] == (np.arange(S)[None, :] // L_SEL))
        .astype(np.float32))
    gw2_pad = jnp.concatenate(
        [g_w2, jnp.zeros((GH, 128 - 3), jnp.float32)], axis=1)
    gb2_pad = jnp.concatenate(
        [g_b2, jnp.full((128 - 3,), NEG, jnp.float32)]).reshape(1, 128)
    gb1_r = g_b1.reshape(1, GH)

    nqb = S // TQ
    out = pl.pallas_call(
        _attn_kernel,
        grid=(nqb,),
        in_specs=[
            pl.BlockSpec((TQ, NH * DK), lambda i: (i, 0)),
            pl.BlockSpec((S, G * DK), lambda i: (0, 0)),
            pl.BlockSpec((S, G * DV), lambda i: (0, 0)),
            pl.BlockSpec((S, G * DK), lambda i: (0, 0)),
            pl.BlockSpec((S, G * DV), lambda i: (0, 0)),
            pl.BlockSpec((NCP, G * DK), lambda i: (0, 0)),
            pl.BlockSpec((NCP, G * DV), lambda i: (0, 0)),
            pl.BlockSpec((NCP, NB), lambda i: (0, 0)),
            pl.BlockSpec((NB, S), lambda i: (0, 0)),
            pl.BlockSpec((DK, GH), lambda i: (0, 0)),
            pl.BlockSpec((1, GH), lambda i: (0, 0)),
            pl.BlockSpec((GH, 128), lambda i: (0, 0)),
            pl.BlockSpec((1, 128), lambda i: (0, 0)),
            pl.BlockSpec((NH * DV, DIM), lambda i: (0, 0)),
        ],
        out_specs=pl.BlockSpec((TQ, DIM), lambda i: (i, 0)),
        out_shape=jax.ShapeDtypeStruct((S, DIM), jnp.float32),
    )(q, ksel, vsel, kwin, vwin, kcsum, vcsum,
      m_pad, expand_all, g_w1, gb1_r, gw2_pad, gb2_pad, W_out)
    return out.reshape(B, S, DIM)
